# Initial kernel scaffold; baseline (speedup 1.0000x reference)
import jax
import jax.numpy as jnp
from jax.experimental import pallas as pl


def kernel(char_indices, lang_indices, char_table, lang_table, W, b):
    raise NotImplementedError("write your pallas kernel here")


# speedup vs baseline: 3.0315x; 3.0315x over previous
import functools

import jax
import jax.numpy as jnp
from jax import lax
from jax.experimental import pallas as pl
from jax.experimental.pallas import tpu as pltpu
from jax.experimental.pallas import tpu_sc as plsc

D = 128
LANG_PAD = 104


def _project_body(char_ref, lang_ref, w_ref, b_ref, cout_ref, lout_ref):
    w = w_ref[...]
    w1 = w[:, :D]
    w2 = w[:, D:]
    cn = (((1,), (1,)), ((), ()))
    cout_ref[...] = (
        lax.dot_general(char_ref[...], w1, cn, preferred_element_type=jnp.float32)
        + b_ref[...]
    )
    lout_ref[...] = lax.dot_general(
        lang_ref[...], w2, cn, preferred_element_type=jnp.float32
    )


def _project(char_table, lang_table_padded, W, b2d):
    n_chars = char_table.shape[0]
    return pl.pallas_call(
        _project_body,
        out_shape=[
            jax.ShapeDtypeStruct((n_chars, D), jnp.float32),
            jax.ShapeDtypeStruct((LANG_PAD, D), jnp.float32),
        ],
    )(char_table, lang_table_padded, W, b2d)


def _make_sc_gather(n_total):
    info = plsc.get_sparse_core_info()
    nw = info.num_cores * info.num_subcores
    per_w = n_total // nw
    ch = 128
    n_ch = per_w // ch
    mesh = plsc.VectorSubcoreMesh(core_axis_name="c", subcore_axis_name="s")

    @functools.partial(
        pl.kernel,
        mesh=mesh,
        out_type=jax.ShapeDtypeStruct((n_total, D), jnp.float32),
        scratch_types=[
            pltpu.VMEM((ch,), jnp.int32),
            pltpu.VMEM((ch,), jnp.int32),
            pltpu.VMEM((ch, D), jnp.float32),
            pltpu.VMEM((ch, D), jnp.float32),
            pltpu.SemaphoreType.DMA,
            pltpu.SemaphoreType.DMA,
        ],
    )
    def sc_gather(cproj_hbm, lproj_hbm, ci_hbm, li_hbm, out_hbm,
                  ia_v, ib_v, ra_v, rb_v, sem_a, sem_b):
        wid = lax.axis_index("s") * info.num_cores + lax.axis_index("c")
        base = wid * per_w

        def chunk(g, carry):
            off = base + g * ch
            pltpu.sync_copy(ci_hbm.at[pl.ds(off, ch)], ia_v)
            pltpu.sync_copy(li_hbm.at[pl.ds(off, ch)], ib_v)
            ca = pltpu.async_copy(cproj_hbm.at[ia_v], ra_v, sem_a)
            cb = pltpu.async_copy(lproj_hbm.at[ib_v], rb_v, sem_b)
            ca.wait()
            cb.wait()

            def row(r, c2):
                for c in range(0, D, 16):
                    ra_v[r, pl.ds(c, 16)] = (
                        ra_v[r, pl.ds(c, 16)] + rb_v[r, pl.ds(c, 16)]
                    )
                return c2

            lax.fori_loop(0, ch, row, 0)
            pltpu.sync_copy(ra_v, out_hbm.at[pl.ds(off, ch)])
            return carry

        lax.fori_loop(0, n_ch, chunk, 0)

    return sc_gather


def kernel(char_indices, lang_indices, char_table, lang_table, W, b):
    B, L = char_indices.shape
    n_total = B * L
    lang_padded = jnp.pad(lang_table, ((0, LANG_PAD - lang_table.shape[0]), (0, 0)))
    cproj, lproj = _project(char_table, lang_padded, W, b.reshape(1, D))
    ci = char_indices.reshape(-1).astype(jnp.int32)
    li = lang_indices.reshape(-1).astype(jnp.int32)
    out = _make_sc_gather(n_total)(cproj, lproj, ci, li)
    return out.reshape(B, L, D)
